# baseline (speedup 1.0000x reference)
import functools

import jax
import jax.numpy as jnp
from jax import lax
from jax.experimental import pallas as pl
from jax.experimental.pallas import tpu as pltpu
from jax.experimental.pallas import tpu_sc as plsc

N = 10000
D = 128
E = 320000
L = 100000

NC = 2
NS = 16
NW = NC * NS

NPAD = 10240
CHUNK = 128
EPAD = 327680
ECH = EPAD // CHUNK
_A0 = 128
_A1 = 32
_PASS = 32
LPAD = 102400
LPT = LPAD // NW

_mesh = plsc.VectorSubcoreMesh(
    core_axis_name="c", subcore_axis_name="s", num_cores=NC, num_subcores=NS)


def _zero_vmem_rows(buf, nrows):
    zeros16 = jnp.zeros((16,), jnp.float32)

    def body(r, _):
        for i in range(D // 16):
            buf[r, pl.ds(i * 16, 16)] = zeros16
        return 0

    lax.fori_loop(0, nrows, body, 0)


_DCH = ECH // NW


@functools.partial(
    pl.kernel,
    out_type=jax.ShapeDtypeStruct((2, NC, NPAD), jnp.float32),
    mesh=_mesh,
    scratch_types=[
        pltpu.VMEM((_DCH, CHUNK), jnp.int32),
        pltpu.VMEM((_DCH, CHUNK), jnp.int32),
        pltpu.VMEM((CHUNK,), jnp.float32),
        pltpu.VMEM((CHUNK,), jnp.float32),
        pltpu.VMEM_SHARED((NPAD,), jnp.float32),
        pltpu.VMEM_SHARED((NPAD,), jnp.float32),
        pltpu.SemaphoreType.DMA,
    ],
)
def _deg_kernel(dst_i_hbm, dst_d_hbm, out_hbm, idx_i_v, idx_d_v, ones_v,
                zeros_v, deg_i_sh, deg_d_sh, sem):
    c = lax.axis_index("c")
    s = lax.axis_index("s")
    wid = c * NS + s
    for i in range(CHUNK // 16):
        ones_v[pl.ds(i * 16, 16)] = jnp.full((16,), 1.0, jnp.float32)
        zeros_v[pl.ds(i * 16, 16)] = jnp.zeros((16,), jnp.float32)
    rows_per_sub = NPAD // NS
    for j in range(rows_per_sub // CHUNK):
        off = s * rows_per_sub + j * CHUNK
        pltpu.sync_copy(zeros_v, deg_i_sh.at[pl.ds(off, CHUNK)])
        pltpu.sync_copy(zeros_v, deg_d_sh.at[pl.ds(off, CHUNK)])
    plsc.subcore_barrier()

    dbase = pl.multiple_of(wid * _DCH, 16)
    pltpu.sync_copy(dst_i_hbm.at[pl.ds(dbase, _DCH)], idx_i_v)
    pltpu.sync_copy(dst_d_hbm.at[pl.ds(dbase, _DCH)], idx_d_v)
    for idx_v, deg_sh in ((idx_i_v, deg_i_sh), (idx_d_v, deg_d_sh)):
        def body(k, _):
            pltpu.sync_copy(ones_v, deg_sh.at[idx_v.at[k]], add=True)
            return 0
        lax.fori_loop(0, _DCH, body, 0)
    plsc.subcore_barrier()

    for j in range(rows_per_sub // CHUNK):
        off = s * rows_per_sub + j * CHUNK
        pltpu.sync_copy(deg_i_sh.at[pl.ds(off, CHUNK)],
                        out_hbm.at[0, c, pl.ds(off, CHUNK)])
        pltpu.sync_copy(deg_d_sh.at[pl.ds(off, CHUNK)],
                        out_hbm.at[1, c, pl.ds(off, CHUNK)])


@functools.partial(
    pl.kernel,
    out_type=jax.ShapeDtypeStruct((NC, NPAD, D), jnp.float32),
    mesh=_mesh,
    scratch_types=[
        pltpu.VMEM((_PASS, CHUNK), jnp.int32),
        pltpu.VMEM((_PASS, CHUNK), jnp.int32),
        pltpu.VMEM((CHUNK,), jnp.int32),
        pltpu.VMEM((CHUNK,), jnp.int32),
        pltpu.VMEM((CHUNK, D), jnp.float32),
        pltpu.VMEM((CHUNK, D), jnp.float32),
        pltpu.VMEM_SHARED((NPAD, D), jnp.float32),
        pltpu.SemaphoreType.DMA,
        pltpu.SemaphoreType.DMA,
    ],
)
def _agg_kernel(g_hbm, src_hbm, dst_hbm, out_hbm, src_v, dst_v, src1_v,
                dst1_v, rows0, rows1, acc_sh, sem0, sem1):
    c = lax.axis_index("c")
    s = lax.axis_index("s")
    rows_per_sub = NPAD // NS
    _zero_vmem_rows(rows0, CHUNK)
    for j in range(rows_per_sub // CHUNK):
        off = s * rows_per_sub + j * CHUNK
        pltpu.sync_copy(rows0, acc_sh.at[pl.ds(off, CHUNK)])
    plsc.subcore_barrier()

    bufs = (rows0, rows1)
    sems = (sem0, sem1)

    @pl.when(c == 0)
    def _fast_core():
        for p in range(_A0 // _PASS):
            pbase = pl.multiple_of(s * _A0 + p * _PASS, _PASS)
            pltpu.sync_copy(src_hbm.at[pl.ds(pbase, _PASS)], src_v)
            pltpu.sync_copy(dst_hbm.at[pl.ds(pbase, _PASS)], dst_v)
            pltpu.async_copy(g_hbm.at[src_v.at[0]], rows0, sem0)
            pltpu.async_copy(g_hbm.at[src_v.at[1]], rows1, sem1)

            def body(j, _):
                k2 = j * 2
                for b in range(2):
                    pltpu.make_async_copy(g_hbm.at[src_v.at[k2 + b]], bufs[b],
                                          sems[b]).wait()
                    pltpu.sync_copy(bufs[b], acc_sh.at[dst_v.at[k2 + b]],
                                    add=True)
                    pltpu.async_copy(g_hbm.at[src_v.at[k2 + b + 2]], bufs[b],
                                     sems[b])
                return 0

            lax.fori_loop(0, _PASS // 2 - 1, body, 0)
            for b in range(2):
                k = _PASS - 2 + b
                pltpu.make_async_copy(g_hbm.at[src_v.at[k]], bufs[b],
                                      sems[b]).wait()
                pltpu.sync_copy(bufs[b], acc_sh.at[dst_v.at[k]], add=True)

    @pl.when(c == 1)
    def _slow_core():
        ebase = NS * _A0 + s * _A1

        def body(k, _):
            ch = ebase + k
            pltpu.sync_copy(src_hbm.at[ch], src1_v)
            pltpu.sync_copy(dst_hbm.at[ch], dst1_v)
            pltpu.async_copy(g_hbm.at[src1_v], rows0, sem0).wait()
            pltpu.sync_copy(rows0, acc_sh.at[dst1_v], add=True)
            return 0

        lax.fori_loop(0, _A1, body, 0)

    plsc.subcore_barrier()
    for j in range(rows_per_sub // CHUNK):
        off = s * rows_per_sub + j * CHUNK
        pltpu.sync_copy(acc_sh.at[pl.ds(off, CHUNK)],
                        out_hbm.at[c, pl.ds(off, CHUNK)])


@functools.partial(
    pl.kernel,
    out_type=jax.ShapeDtypeStruct((2, LPAD, D), jnp.float32),
    mesh=_mesh,
    scratch_types=[
        pltpu.VMEM((CHUNK,), jnp.int32),
        pltpu.VMEM((CHUNK, D), jnp.float32),
        pltpu.SemaphoreType.DMA,
    ],
)
def _decode_gather_kernel(z_hbm, el_hbm, out_hbm, idx_v, rows_v, sem):
    c = lax.axis_index("c")
    s = lax.axis_index("s")
    base = (c * NS + s) * LPT
    for side in (0, 1):
        def body(k, _):
            off = base + k * CHUNK
            pltpu.sync_copy(el_hbm.at[side, pl.ds(off, CHUNK)], idx_v)
            pltpu.async_copy(z_hbm.at[idx_v], rows_v, sem).wait()
            pltpu.sync_copy(rows_v, out_hbm.at[side, pl.ds(off, CHUNK)])
            return 0

        lax.fori_loop(0, LPT // CHUNK, body, 0)


_RB = 1024


def _encode_body(x_ref, w_ref, di0, di1, dd0, dd1, gi_ref, gd_ref):
    h = jnp.dot(x_ref[...], w_ref[...], preferred_element_type=jnp.float32)
    dinv_i = lax.rsqrt(di0[...] + di1[...] + 1.0)
    dinv_d = lax.rsqrt(dd0[...] + dd1[...] + 1.0)
    gi_ref[...] = h * dinv_i
    gd_ref[...] = h * dinv_d


def _encode(x_pad, w, deg_i0, deg_i1, deg_d0, deg_d1):
    nb = NPAD // _RB
    col = pl.BlockSpec((_RB, 1), lambda i: (i, 0))
    return pl.pallas_call(
        _encode_body,
        grid=(nb,),
        in_specs=[
            pl.BlockSpec((_RB, D), lambda i: (i, 0)),
            pl.BlockSpec((D, D), lambda i: (0, 0)),
            col, col, col, col,
        ],
        out_specs=[pl.BlockSpec((_RB, D), lambda i: (i, 0))] * 2,
        out_shape=[jax.ShapeDtypeStruct((NPAD, D), jnp.float32)] * 2,
    )(x_pad, w, deg_i0, deg_i1, deg_d0, deg_d1)


def _combine_body(acci_ref, accd_ref, gi_ref, gd_ref, di0, di1, dd0, dd1,
                  b_ref, z_ref):
    dinv_i = lax.rsqrt(di0[...] + di1[...] + 1.0)
    dinv_d = lax.rsqrt(dd0[...] + dd1[...] + 1.0)
    b = b_ref[...]
    zi = jnp.maximum(
        dinv_i * (acci_ref[0] + acci_ref[1] + gi_ref[...]) + b, 0.0)
    zd = jnp.maximum(
        dinv_d * (accd_ref[0] + accd_ref[1] + gd_ref[...]) + b, 0.0)
    z_ref[...] = zi + zd


def _combine(acc_i, acc_d, g_i, g_d, deg_i0, deg_i1, deg_d0, deg_d1, b_conv):
    nb = NPAD // _RB
    col = pl.BlockSpec((_RB, 1), lambda i: (i, 0))
    row_spec = pl.BlockSpec((_RB, D), lambda i: (i, 0))
    acc_spec = pl.BlockSpec((NC, _RB, D), lambda i: (0, i, 0))
    return pl.pallas_call(
        _combine_body,
        grid=(nb,),
        in_specs=[
            acc_spec, acc_spec, row_spec, row_spec,
            col, col, col, col,
            pl.BlockSpec((1, D), lambda i: (0, 0)),
        ],
        out_specs=row_spec,
        out_shape=jax.ShapeDtypeStruct((NPAD, D), jnp.float32),
    )(acc_i, acc_d, g_i, g_d, deg_i0, deg_i1, deg_d0, deg_d1, b_conv)


def _decoder_body(zz_ref, w1_ref, b1_ref, w2_ref, b2_ref, o_ref):
    zi = zz_ref[0]
    zj = zz_ref[1]
    w1 = w1_ref[...]
    pre = (jnp.dot(zi, w1[0:D], preferred_element_type=jnp.float32)
           + jnp.dot(zj, w1[D:2 * D], preferred_element_type=jnp.float32)
           + jnp.dot(jnp.abs(zi - zj), w1[2 * D:3 * D],
                     preferred_element_type=jnp.float32))
    hid = jnp.maximum(pre + b1_ref[...], 0.0)
    o = jnp.sum(hid * w2_ref[...], axis=1, keepdims=True) + b2_ref[...]
    o_ref[...] = o


def _decoder(zz, w1, b1, w2_row, b2):
    nb = LPAD // _RB
    return pl.pallas_call(
        _decoder_body,
        grid=(nb,),
        in_specs=[
            pl.BlockSpec((2, _RB, D), lambda i: (0, i, 0)),
            pl.BlockSpec((3 * D, D), lambda i: (0, 0)),
            pl.BlockSpec((1, D), lambda i: (0, 0)),
            pl.BlockSpec((1, D), lambda i: (0, 0)),
            pl.BlockSpec((1, 1), lambda i: (0, 0)),
        ],
        out_specs=pl.BlockSpec((_RB, 1), lambda i: (i, 0)),
        out_shape=jax.ShapeDtypeStruct((LPAD, 1), jnp.float32),
    )(zz, w1, b1, w2_row, b2)


def kernel(x, edge_index, edge_index_D, edge_label_index, W_conv, b_conv,
           W1, b1, W2, b2):
    pad_e = EPAD - E
    pad_src = jnp.zeros((pad_e,), jnp.int32)
    pad_dst = jnp.full((pad_e,), N, jnp.int32)
    esh = (ECH, CHUNK)
    src_i = jnp.concatenate([edge_index[0], pad_src]).reshape(esh)
    dst_i = jnp.concatenate([edge_index[1], pad_dst]).reshape(esh)
    src_d = jnp.concatenate([edge_index_D[0], pad_src]).reshape(esh)
    dst_d = jnp.concatenate([edge_index_D[1], pad_dst]).reshape(esh)
    pad_l = jnp.zeros((LPAD - L,), jnp.int32)
    el = jnp.stack([
        jnp.concatenate([edge_label_index[0], pad_l]),
        jnp.concatenate([edge_label_index[1], pad_l]),
    ])

    x_pad = jnp.pad(x, ((0, NPAD - N), (0, 0)))

    deg = _deg_kernel(dst_i, dst_d)
    deg_i0 = deg[0, 0].reshape(NPAD, 1)
    deg_i1 = deg[0, 1].reshape(NPAD, 1)
    deg_d0 = deg[1, 0].reshape(NPAD, 1)
    deg_d1 = deg[1, 1].reshape(NPAD, 1)

    g_i, g_d = _encode(x_pad, W_conv, deg_i0, deg_i1, deg_d0, deg_d1)

    acc_i = _agg_kernel(g_i, src_i, dst_i)
    acc_d = _agg_kernel(g_d, src_d, dst_d)

    z = _combine(acc_i, acc_d, g_i, g_d, deg_i0, deg_i1, deg_d0, deg_d1,
                 b_conv.reshape(1, D))

    zz = _decode_gather_kernel(z, el)

    out = _decoder(zz, W1, b1.reshape(1, D), W2.reshape(1, D),
                   b2.reshape(1, 1))
    return out.reshape(LPAD)[:L]


# speedup vs baseline: 1.0270x; 1.0270x over previous
import functools

import jax
import jax.numpy as jnp
from jax import lax
from jax.experimental import pallas as pl
from jax.experimental.pallas import tpu as pltpu
from jax.experimental.pallas import tpu_sc as plsc

N = 10000
D = 128
E = 320000
L = 100000

NC = 2
NS = 16
NW = NC * NS

NPAD = 10240
CHUNK = 128
EPAD = 327680
ECH = EPAD // CHUNK
_A0 = 144
_A1 = 16
_PASS = 16
LPAD = 102400
_L0 = 35
_L1 = 15

_mesh = plsc.VectorSubcoreMesh(
    core_axis_name="c", subcore_axis_name="s", num_cores=NC, num_subcores=NS)


def _zero_vmem_rows(buf, nrows):
    zeros16 = jnp.zeros((16,), jnp.float32)

    def body(r, _):
        for i in range(D // 16):
            buf[r, pl.ds(i * 16, 16)] = zeros16
        return 0

    lax.fori_loop(0, nrows, body, 0)


_DCH = ECH // NW


@functools.partial(
    pl.kernel,
    out_type=jax.ShapeDtypeStruct((2, NC, NPAD), jnp.float32),
    mesh=_mesh,
    scratch_types=[
        pltpu.VMEM((_DCH, CHUNK), jnp.int32),
        pltpu.VMEM((_DCH, CHUNK), jnp.int32),
        pltpu.VMEM((CHUNK,), jnp.float32),
        pltpu.VMEM((CHUNK,), jnp.float32),
        pltpu.VMEM_SHARED((NPAD,), jnp.float32),
        pltpu.VMEM_SHARED((NPAD,), jnp.float32),
        pltpu.SemaphoreType.DMA,
    ],
)
def _deg_kernel(dst_i_hbm, dst_d_hbm, out_hbm, idx_i_v, idx_d_v, ones_v,
                zeros_v, deg_i_sh, deg_d_sh, sem):
    c = lax.axis_index("c")
    s = lax.axis_index("s")
    wid = c * NS + s
    for i in range(CHUNK // 16):
        ones_v[pl.ds(i * 16, 16)] = jnp.full((16,), 1.0, jnp.float32)
        zeros_v[pl.ds(i * 16, 16)] = jnp.zeros((16,), jnp.float32)
    rows_per_sub = NPAD // NS
    for j in range(rows_per_sub // CHUNK):
        off = s * rows_per_sub + j * CHUNK
        pltpu.sync_copy(zeros_v, deg_i_sh.at[pl.ds(off, CHUNK)])
        pltpu.sync_copy(zeros_v, deg_d_sh.at[pl.ds(off, CHUNK)])
    plsc.subcore_barrier()

    dbase = pl.multiple_of(wid * _DCH, 16)
    pltpu.sync_copy(dst_i_hbm.at[pl.ds(dbase, _DCH)], idx_i_v)
    pltpu.sync_copy(dst_d_hbm.at[pl.ds(dbase, _DCH)], idx_d_v)
    for idx_v, deg_sh in ((idx_i_v, deg_i_sh), (idx_d_v, deg_d_sh)):
        def body(k, _):
            pltpu.sync_copy(ones_v, deg_sh.at[idx_v.at[k]], add=True)
            return 0
        lax.fori_loop(0, _DCH, body, 0)
    plsc.subcore_barrier()

    for j in range(rows_per_sub // CHUNK):
        off = s * rows_per_sub + j * CHUNK
        pltpu.sync_copy(deg_i_sh.at[pl.ds(off, CHUNK)],
                        out_hbm.at[0, c, pl.ds(off, CHUNK)])
        pltpu.sync_copy(deg_d_sh.at[pl.ds(off, CHUNK)],
                        out_hbm.at[1, c, pl.ds(off, CHUNK)])


@functools.partial(
    pl.kernel,
    out_type=jax.ShapeDtypeStruct((NC, NPAD, D), jnp.float32),
    mesh=_mesh,
    scratch_types=[
        pltpu.VMEM((_PASS, CHUNK), jnp.int32),
        pltpu.VMEM((_PASS, CHUNK), jnp.int32),
        pltpu.VMEM((CHUNK,), jnp.int32),
        pltpu.VMEM((CHUNK,), jnp.int32),
        pltpu.VMEM((CHUNK, D), jnp.float32),
        pltpu.VMEM((CHUNK, D), jnp.float32),
        pltpu.VMEM_SHARED((NPAD, D), jnp.float32),
        pltpu.SemaphoreType.DMA,
        pltpu.SemaphoreType.DMA,
    ],
)
def _agg_kernel(g_hbm, src_hbm, dst_hbm, out_hbm, src_v, dst_v, src1_v,
                dst1_v, rows0, rows1, acc_sh, sem0, sem1):
    c = lax.axis_index("c")
    s = lax.axis_index("s")
    rows_per_sub = NPAD // NS
    _zero_vmem_rows(rows0, CHUNK)
    for j in range(rows_per_sub // CHUNK):
        off = s * rows_per_sub + j * CHUNK
        pltpu.sync_copy(rows0, acc_sh.at[pl.ds(off, CHUNK)])
    plsc.subcore_barrier()

    bufs = (rows0, rows1)
    sems = (sem0, sem1)

    @pl.when(c == 0)
    def _fast_core():
        for p in range(_A0 // _PASS):
            pbase = pl.multiple_of(s * _A0 + p * _PASS, _PASS)
            pltpu.sync_copy(src_hbm.at[pl.ds(pbase, _PASS)], src_v)
            pltpu.sync_copy(dst_hbm.at[pl.ds(pbase, _PASS)], dst_v)
            pltpu.async_copy(g_hbm.at[src_v.at[0]], rows0, sem0)
            pltpu.async_copy(g_hbm.at[src_v.at[1]], rows1, sem1)

            def body(j, _):
                k2 = j * 2
                for b in range(2):
                    pltpu.make_async_copy(g_hbm.at[src_v.at[k2 + b]], bufs[b],
                                          sems[b]).wait()
                    pltpu.sync_copy(bufs[b], acc_sh.at[dst_v.at[k2 + b]],
                                    add=True)
                    pltpu.async_copy(g_hbm.at[src_v.at[k2 + b + 2]], bufs[b],
                                     sems[b])
                return 0

            lax.fori_loop(0, _PASS // 2 - 1, body, 0)
            for b in range(2):
                k = _PASS - 2 + b
                pltpu.make_async_copy(g_hbm.at[src_v.at[k]], bufs[b],
                                      sems[b]).wait()
                pltpu.sync_copy(bufs[b], acc_sh.at[dst_v.at[k]], add=True)

    @pl.when(c == 1)
    def _slow_core():
        ebase = NS * _A0 + s * _A1

        def body(k, _):
            ch = ebase + k
            pltpu.sync_copy(src_hbm.at[ch], src1_v)
            pltpu.sync_copy(dst_hbm.at[ch], dst1_v)
            pltpu.async_copy(g_hbm.at[src1_v], rows0, sem0).wait()
            pltpu.sync_copy(rows0, acc_sh.at[dst1_v], add=True)
            return 0

        lax.fori_loop(0, _A1, body, 0)

    plsc.subcore_barrier()
    for j in range(rows_per_sub // CHUNK):
        off = s * rows_per_sub + j * CHUNK
        pltpu.sync_copy(acc_sh.at[pl.ds(off, CHUNK)],
                        out_hbm.at[c, pl.ds(off, CHUNK)])


@functools.partial(
    pl.kernel,
    out_type=jax.ShapeDtypeStruct((2, LPAD, D), jnp.float32),
    mesh=_mesh,
    scratch_types=[
        pltpu.VMEM((CHUNK,), jnp.int32),
        pltpu.VMEM((CHUNK, D), jnp.float32),
        pltpu.SemaphoreType.DMA,
    ],
)
def _decode_gather_kernel(z_hbm, el_hbm, out_hbm, idx_v, rows_v, sem):
    c = lax.axis_index("c")
    s = lax.axis_index("s")
    base = jnp.where(c == 0, s * _L0, NS * _L0 + s * _L1)
    cnt = jnp.where(c == 0, _L0, _L1)
    for side in (0, 1):
        def body(k, _):
            off = pl.multiple_of((base + k) * CHUNK, CHUNK)
            pltpu.sync_copy(el_hbm.at[side, pl.ds(off, CHUNK)], idx_v)
            pltpu.async_copy(z_hbm.at[idx_v], rows_v, sem).wait()
            pltpu.sync_copy(rows_v, out_hbm.at[side, pl.ds(off, CHUNK)])
            return 0

        lax.fori_loop(0, cnt, body, 0)


_RB = 1024


def _encode_body(x_ref, w_ref, di0, di1, dd0, dd1, gi_ref, gd_ref):
    h = jnp.dot(x_ref[...], w_ref[...], preferred_element_type=jnp.float32)
    dinv_i = lax.rsqrt(di0[...] + di1[...] + 1.0)
    dinv_d = lax.rsqrt(dd0[...] + dd1[...] + 1.0)
    gi_ref[...] = h * dinv_i
    gd_ref[...] = h * dinv_d


def _encode(x_pad, w, deg_i0, deg_i1, deg_d0, deg_d1):
    nb = NPAD // _RB
    col = pl.BlockSpec((_RB, 1), lambda i: (i, 0))
    return pl.pallas_call(
        _encode_body,
        grid=(nb,),
        in_specs=[
            pl.BlockSpec((_RB, D), lambda i: (i, 0)),
            pl.BlockSpec((D, D), lambda i: (0, 0)),
            col, col, col, col,
        ],
        out_specs=[pl.BlockSpec((_RB, D), lambda i: (i, 0))] * 2,
        out_shape=[jax.ShapeDtypeStruct((NPAD, D), jnp.float32)] * 2,
    )(x_pad, w, deg_i0, deg_i1, deg_d0, deg_d1)


def _combine_body(acci_ref, accd_ref, gi_ref, gd_ref, di0, di1, dd0, dd1,
                  b_ref, z_ref):
    dinv_i = lax.rsqrt(di0[...] + di1[...] + 1.0)
    dinv_d = lax.rsqrt(dd0[...] + dd1[...] + 1.0)
    b = b_ref[...]
    zi = jnp.maximum(
        dinv_i * (acci_ref[0] + acci_ref[1] + gi_ref[...]) + b, 0.0)
    zd = jnp.maximum(
        dinv_d * (accd_ref[0] + accd_ref[1] + gd_ref[...]) + b, 0.0)
    z_ref[...] = zi + zd


def _combine(acc_i, acc_d, g_i, g_d, deg_i0, deg_i1, deg_d0, deg_d1, b_conv):
    nb = NPAD // _RB
    col = pl.BlockSpec((_RB, 1), lambda i: (i, 0))
    row_spec = pl.BlockSpec((_RB, D), lambda i: (i, 0))
    acc_spec = pl.BlockSpec((NC, _RB, D), lambda i: (0, i, 0))
    return pl.pallas_call(
        _combine_body,
        grid=(nb,),
        in_specs=[
            acc_spec, acc_spec, row_spec, row_spec,
            col, col, col, col,
            pl.BlockSpec((1, D), lambda i: (0, 0)),
        ],
        out_specs=row_spec,
        out_shape=jax.ShapeDtypeStruct((NPAD, D), jnp.float32),
    )(acc_i, acc_d, g_i, g_d, deg_i0, deg_i1, deg_d0, deg_d1, b_conv)


def _decoder_body(zz_ref, w1_ref, b1_ref, w2_ref, b2_ref, o_ref):
    zi = zz_ref[0]
    zj = zz_ref[1]
    w1 = w1_ref[...]
    pre = (jnp.dot(zi, w1[0:D], preferred_element_type=jnp.float32)
           + jnp.dot(zj, w1[D:2 * D], preferred_element_type=jnp.float32)
           + jnp.dot(jnp.abs(zi - zj), w1[2 * D:3 * D],
                     preferred_element_type=jnp.float32))
    hid = jnp.maximum(pre + b1_ref[...], 0.0)
    o = jnp.sum(hid * w2_ref[...], axis=1, keepdims=True) + b2_ref[...]
    o_ref[...] = o


def _decoder(zz, w1, b1, w2_row, b2):
    nb = LPAD // _RB
    return pl.pallas_call(
        _decoder_body,
        grid=(nb,),
        in_specs=[
            pl.BlockSpec((2, _RB, D), lambda i: (0, i, 0)),
            pl.BlockSpec((3 * D, D), lambda i: (0, 0)),
            pl.BlockSpec((1, D), lambda i: (0, 0)),
            pl.BlockSpec((1, D), lambda i: (0, 0)),
            pl.BlockSpec((1, 1), lambda i: (0, 0)),
        ],
        out_specs=pl.BlockSpec((_RB, 1), lambda i: (i, 0)),
        out_shape=jax.ShapeDtypeStruct((LPAD, 1), jnp.float32),
    )(zz, w1, b1, w2_row, b2)


def kernel(x, edge_index, edge_index_D, edge_label_index, W_conv, b_conv,
           W1, b1, W2, b2):
    pad_e = EPAD - E
    pad_src = jnp.zeros((pad_e,), jnp.int32)
    pad_dst = jnp.full((pad_e,), N, jnp.int32)
    esh = (ECH, CHUNK)
    src_i = jnp.concatenate([edge_index[0], pad_src]).reshape(esh)
    dst_i = jnp.concatenate([edge_index[1], pad_dst]).reshape(esh)
    src_d = jnp.concatenate([edge_index_D[0], pad_src]).reshape(esh)
    dst_d = jnp.concatenate([edge_index_D[1], pad_dst]).reshape(esh)
    pad_l = jnp.zeros((LPAD - L,), jnp.int32)
    el = jnp.stack([
        jnp.concatenate([edge_label_index[0], pad_l]),
        jnp.concatenate([edge_label_index[1], pad_l]),
    ])

    x_pad = jnp.pad(x, ((0, NPAD - N), (0, 0)))

    deg = _deg_kernel(dst_i, dst_d)
    deg_i0 = deg[0, 0].reshape(NPAD, 1)
    deg_i1 = deg[0, 1].reshape(NPAD, 1)
    deg_d0 = deg[1, 0].reshape(NPAD, 1)
    deg_d1 = deg[1, 1].reshape(NPAD, 1)

    g_i, g_d = _encode(x_pad, W_conv, deg_i0, deg_i1, deg_d0, deg_d1)

    acc_i = _agg_kernel(g_i, src_i, dst_i)
    acc_d = _agg_kernel(g_d, src_d, dst_d)

    z = _combine(acc_i, acc_d, g_i, g_d, deg_i0, deg_i1, deg_d0, deg_d1,
                 b_conv.reshape(1, D))

    zz = _decode_gather_kernel(z, el)

    out = _decoder(zz, W1, b1.reshape(1, D), W2.reshape(1, D),
                   b2.reshape(1, 1))
    return out.reshape(LPAD)[:L]
